# Initial kernel scaffold; baseline (speedup 1.0000x reference)
import jax
import jax.numpy as jnp
from jax.experimental import pallas as pl


def kernel(indices, codebook):
    raise NotImplementedError("write your pallas kernel here")


# speedup vs baseline: 11.2912x; 11.2912x over previous
import functools

import numpy as np
import jax
import jax.numpy as jnp
from jax import lax
from jax.experimental import pallas as pl
from jax.experimental.pallas import tpu as pltpu
from jax.experimental.pallas import tpu_sc as plsc

B = 16384
T = 50
D = 64
NC = 2
NS = 16
NW = NC * NS
BPW = B // NW
TPW = BPW * T
G = 128
NCH = TPW // G

_SIDX = (np.arange(TPW, dtype=np.int32) // T).reshape(NCH, G)

_mesh = plsc.VectorSubcoreMesh(core_axis_name="c", subcore_axis_name="s")


@functools.partial(
    pl.kernel,
    mesh=_mesh,
    out_type=jax.ShapeDtypeStruct((B, D), jnp.float32),
    compiler_params=pltpu.CompilerParams(use_tc_tiling_on_sc=False),
    scratch_types=[
        pltpu.VMEM((NCH, G), jnp.int32),
        pltpu.VMEM((NCH, G), jnp.int32),
        pltpu.VMEM((G, D), jnp.float32),
        pltpu.VMEM((BPW, D), jnp.float32),
        pltpu.VMEM_SHARED((NS, BPW, D), jnp.float32),
    ],
)
def _emb_sum(cb_hbm, idx_hbm, sidx_hbm, out_hbm, idx_v, sidx_v, buf_v,
             zero_v, acc_sh):
    sid = lax.axis_index("s")
    wid = sid * NC + lax.axis_index("c")
    pltpu.sync_copy(idx_hbm.at[wid], idx_v)
    pltpu.sync_copy(sidx_hbm, sidx_v)

    zeros = jnp.zeros((16,), jnp.float32)

    def zrow(r, carry):
        for dd in range(D // 16):
            zero_v[r, pl.ds(dd * 16, 16)] = zeros
        return carry

    lax.fori_loop(0, BPW, zrow, 0)
    pltpu.sync_copy(zero_v, acc_sh.at[sid])

    def chunk(c, carry):
        pltpu.sync_copy(cb_hbm.at[idx_v.at[c]], buf_v)
        pltpu.sync_copy(buf_v, acc_sh.at[sid].at[sidx_v.at[c]], add=True)
        return carry

    lax.fori_loop(0, NCH, chunk, 0)

    pltpu.sync_copy(acc_sh.at[sid], out_hbm.at[pl.ds(wid * BPW, BPW)])


def kernel(indices, codebook):
    idx3 = indices.astype(jnp.int32).reshape(NW, NCH, G)
    return _emb_sum(codebook, idx3, jnp.asarray(_SIDX))


# speedup vs baseline: 14.3717x; 1.2728x over previous
import functools

import numpy as np
import jax
import jax.numpy as jnp
from jax import lax
from jax.experimental import pallas as pl
from jax.experimental.pallas import tpu as pltpu
from jax.experimental.pallas import tpu_sc as plsc

B = 16384
T = 50
D = 64
NC = 2
NS = 16
NW = NC * NS
BPW = B // NW
G = 128
KQ = BPW // G
NQ = T * KQ

_mesh = plsc.VectorSubcoreMesh(core_axis_name="c", subcore_axis_name="s")


@functools.partial(
    pl.kernel,
    mesh=_mesh,
    out_type=jax.ShapeDtypeStruct((B, D), jnp.float32),
    compiler_params=pltpu.CompilerParams(use_tc_tiling_on_sc=False),
    scratch_types=[
        pltpu.VMEM((NQ, G), jnp.int32),
        pltpu.VMEM((BPW, D), jnp.float32),
    ],
)
def _emb_sum(cb_hbm, idx_hbm, out_hbm, idx_v, acc_v):
    sid = lax.axis_index("s")
    wid = sid * NC + lax.axis_index("c")
    pltpu.sync_copy(idx_hbm.at[wid], idx_v)

    zeros = jnp.zeros((16,), jnp.float32)

    def zrow(r, carry):
        for dd in range(D // 16):
            acc_v[r, pl.ds(dd * 16, 16)] = zeros
        return carry

    lax.fori_loop(0, BPW, zrow, 0)

    def step(q, carry):
        k = lax.rem(q, KQ)
        pltpu.sync_copy(
            cb_hbm.at[idx_v.at[q]], acc_v.at[pl.ds(k * G, G)], add=True)
        return carry

    lax.fori_loop(0, NQ, step, 0)

    pltpu.sync_copy(acc_v, out_hbm.at[pl.ds(wid * BPW, BPW)])


def kernel(indices, codebook):
    idx = indices.astype(jnp.int32).reshape(NW, KQ, G, T)
    idx = idx.transpose(0, 3, 1, 2).reshape(NW, NQ, G)
    return _emb_sum(codebook, idx)


# speedup vs baseline: 24.1004x; 1.6769x over previous
import functools

import numpy as np
import jax
import jax.numpy as jnp
from jax import lax
from jax.experimental import pallas as pl
from jax.experimental.pallas import tpu as pltpu
from jax.experimental.pallas import tpu_sc as plsc

B = 16384
T = 50
D = 64
NC = 2
NS = 16
NW = NC * NS
BPW = B // NW
G = 128
KQ = BPW // G
NQ = T * KQ
WIN = 16

_mesh = plsc.VectorSubcoreMesh(core_axis_name="c", subcore_axis_name="s")


@functools.partial(
    pl.kernel,
    mesh=_mesh,
    out_type=jax.ShapeDtypeStruct((B, D), jnp.float32),
    compiler_params=pltpu.CompilerParams(use_tc_tiling_on_sc=False),
    scratch_types=[
        pltpu.VMEM((NQ, G), jnp.int32),
        pltpu.VMEM((BPW, D), jnp.float32),
        pltpu.SemaphoreType.DMA,
    ],
)
def _emb_sum(cb_hbm, idx_hbm, out_hbm, idx_v, acc_v, gsem):
    sid = lax.axis_index("s")
    wid = sid * NC + lax.axis_index("c")
    pltpu.sync_copy(idx_hbm.at[wid], idx_v)

    zeros = jnp.zeros((16,), jnp.float32)

    def zrow(r, carry):
        for dd in range(D // 16):
            acc_v[r, pl.ds(dd * 16, 16)] = zeros
        return carry

    lax.fori_loop(0, BPW, zrow, 0)

    def step(q, carry):
        k = lax.rem(q, KQ)
        pltpu.async_copy(
            cb_hbm.at[idx_v.at[q]], acc_v.at[pl.ds(k * G, G)], gsem,
            add=True)

        @pl.when(q >= WIN)
        def _():
            pltpu.make_async_copy(
                cb_hbm.at[idx_v.at[q - WIN]], acc_v.at[pl.ds(k * G, G)],
                gsem).wait()

        return carry

    lax.fori_loop(0, NQ, step, 0)
    for j in range(WIN):
        q = NQ - WIN + j
        k = q % KQ
        pltpu.make_async_copy(
            cb_hbm.at[idx_v.at[q]], acc_v.at[pl.ds(k * G, G)], gsem).wait()

    pltpu.sync_copy(acc_v, out_hbm.at[pl.ds(wid * BPW, BPW)])


def kernel(indices, codebook):
    idx = indices.astype(jnp.int32).reshape(NW, KQ, G, T)
    idx = idx.transpose(0, 3, 1, 2).reshape(NW, NQ, G)
    return _emb_sum(codebook, idx)


# speedup vs baseline: 24.1013x; 1.0000x over previous
import functools

import numpy as np
import jax
import jax.numpy as jnp
from jax import lax
from jax.experimental import pallas as pl
from jax.experimental.pallas import tpu as pltpu
from jax.experimental.pallas import tpu_sc as plsc

B = 16384
T = 50
D = 64
NC = 2
NS = 16
NW = NC * NS
BPW = B // NW
G = 128
KQ = BPW // G
NQ = T * KQ
WIN = 32

_mesh = plsc.VectorSubcoreMesh(core_axis_name="c", subcore_axis_name="s")


@functools.partial(
    pl.kernel,
    mesh=_mesh,
    out_type=jax.ShapeDtypeStruct((B, D), jnp.float32),
    compiler_params=pltpu.CompilerParams(use_tc_tiling_on_sc=False),
    scratch_types=[
        pltpu.VMEM((NQ, G), jnp.int32),
        pltpu.VMEM((BPW, D), jnp.float32),
        pltpu.SemaphoreType.DMA,
    ],
)
def _emb_sum(cb_hbm, idx_hbm, out_hbm, idx_v, acc_v, gsem):
    sid = lax.axis_index("s")
    wid = sid * NC + lax.axis_index("c")
    pltpu.sync_copy(idx_hbm.at[wid], idx_v)

    zeros = jnp.zeros((16,), jnp.float32)

    def zrow(r, carry):
        for dd in range(D // 16):
            acc_v[r, pl.ds(dd * 16, 16)] = zeros
        return carry

    lax.fori_loop(0, BPW, zrow, 0)

    def step(q, carry):
        k = lax.rem(q, KQ)
        pltpu.async_copy(
            cb_hbm.at[idx_v.at[q]], acc_v.at[pl.ds(k * G, G)], gsem,
            add=True)

        @pl.when(q >= WIN)
        def _():
            pltpu.make_async_copy(
                cb_hbm.at[idx_v.at[q - WIN]], acc_v.at[pl.ds(k * G, G)],
                gsem).wait()

        return carry

    lax.fori_loop(0, NQ, step, 0)
    for j in range(WIN):
        q = NQ - WIN + j
        k = q % KQ
        pltpu.make_async_copy(
            cb_hbm.at[idx_v.at[q]], acc_v.at[pl.ds(k * G, G)], gsem).wait()

    pltpu.sync_copy(acc_v, out_hbm.at[pl.ds(wid * BPW, BPW)])


def kernel(indices, codebook):
    idx = indices.astype(jnp.int32).reshape(NW, KQ, G, T)
    idx = idx.transpose(0, 3, 1, 2).reshape(NW, NQ, G)
    return _emb_sum(codebook, idx)


# speedup vs baseline: 24.3128x; 1.0088x over previous
import functools

import numpy as np
import jax
import jax.numpy as jnp
from jax import lax
from jax.experimental import pallas as pl
from jax.experimental.pallas import tpu as pltpu
from jax.experimental.pallas import tpu_sc as plsc

B = 16384
T = 50
D = 64
NC = 2
NS = 16
NW = NC * NS
BPW = B // NW
G = 512
KQ = BPW // G
NQ = T * KQ
WIN = 8

_mesh = plsc.VectorSubcoreMesh(core_axis_name="c", subcore_axis_name="s")


@functools.partial(
    pl.kernel,
    mesh=_mesh,
    out_type=jax.ShapeDtypeStruct((B, D), jnp.float32),
    compiler_params=pltpu.CompilerParams(use_tc_tiling_on_sc=False),
    scratch_types=[
        pltpu.VMEM((NQ, G), jnp.int32),
        pltpu.VMEM((BPW, D), jnp.float32),
        pltpu.SemaphoreType.DMA,
    ],
)
def _emb_sum(cb_hbm, idx_hbm, out_hbm, idx_v, acc_v, gsem):
    sid = lax.axis_index("s")
    wid = sid * NC + lax.axis_index("c")
    pltpu.sync_copy(idx_hbm.at[wid], idx_v)

    zeros = jnp.zeros((16,), jnp.float32)

    def zrow(r, carry):
        for dd in range(D // 16):
            acc_v[r, pl.ds(dd * 16, 16)] = zeros
        return carry

    lax.fori_loop(0, BPW, zrow, 0)

    def step(q, carry):
        k = lax.rem(q, KQ)
        pltpu.async_copy(
            cb_hbm.at[idx_v.at[q]], acc_v.at[pl.ds(k * G, G)], gsem,
            add=True)

        @pl.when(q >= WIN)
        def _():
            pltpu.make_async_copy(
                cb_hbm.at[idx_v.at[q - WIN]], acc_v.at[pl.ds(k * G, G)],
                gsem).wait()

        return carry

    lax.fori_loop(0, NQ, step, 0)
    for j in range(WIN):
        q = NQ - WIN + j
        k = q % KQ
        pltpu.make_async_copy(
            cb_hbm.at[idx_v.at[q]], acc_v.at[pl.ds(k * G, G)], gsem).wait()

    pltpu.sync_copy(acc_v, out_hbm.at[pl.ds(wid * BPW, BPW)])


def kernel(indices, codebook):
    idx = indices.astype(jnp.int32).reshape(NW, KQ, G, T)
    idx = idx.transpose(0, 3, 1, 2).reshape(NW, NQ, G)
    return _emb_sum(codebook, idx)
